# baseline (speedup 1.0000x reference)
import functools

import jax
import jax.numpy as jnp
from jax import lax
from jax.experimental import pallas as pl
from jax.experimental.pallas import tpu as pltpu
from jax.experimental.pallas import tpu_sc as plsc

_NN = 64
_B = 64
_F = 657
_EP = 4096
_NW = 32
_EPW = _EP // _NW
_KN = 8
_INV = 0.9999950000374997


_sc_mesh = plsc.VectorSubcoreMesh(core_axis_name="c", subcore_axis_name="s")


@functools.partial(
    pl.kernel,
    mesh=_sc_mesh,
    out_type=[
        jax.ShapeDtypeStruct((_NW, _NN * _NN), jnp.float32),
        jax.ShapeDtypeStruct((_NW, _NN * _NN), jnp.float32),
    ],
    scratch_types=[
        pltpu.VMEM((_EPW,), jnp.int32),
        pltpu.VMEM((_EPW,), jnp.int32),
        pltpu.VMEM((_NN * _NN,), jnp.float32),
        pltpu.VMEM((_NN * _NN,), jnp.float32),
    ],
    compiler_params=pltpu.CompilerParams(needs_layout_passes=False),
)
def _edge_hist(src_hbm, dst_hbm, out_hbm, outT_hbm, src_v, dst_v, cnt_v, cntT_v):
    c = lax.axis_index("c")
    s = lax.axis_index("s")
    wid = s * 2 + c
    base = wid * _EPW
    pltpu.sync_copy(src_hbm.at[pl.ds(base, _EPW)], src_v)
    pltpu.sync_copy(dst_hbm.at[pl.ds(base, _EPW)], dst_v)

    zeros16 = jnp.zeros((16,), jnp.float32)

    def _zero(i, carry):
        cnt_v[pl.ds(i * 16, 16)] = zeros16
        cntT_v[pl.ds(i * 16, 16)] = zeros16
        return carry

    lax.fori_loop(0, (_NN * _NN) // 16, _zero, 0)

    ones16 = jnp.ones((16,), jnp.float32)

    def _scatter(j, carry):
        sv = src_v[pl.ds(j * 16, 16)]
        dv = dst_v[pl.ds(j * 16, 16)]
        plsc.addupdate_scatter(cnt_v, [sv * _NN + dv], ones16)
        plsc.addupdate_scatter(cntT_v, [dv * _NN + sv], ones16)
        return carry

    lax.fori_loop(0, _EPW // 16, _scatter, 0)

    pltpu.sync_copy(cnt_v, out_hbm.at[wid])
    pltpu.sync_copy(cntT_v, outT_hbm.at[wid])


def _fused_body(cnt_ref, cntT_ref, adj_ref, x0_ref, w0_ref, w1_ref, w2_ref,
                xr_ref, wf_ref, cheb_ref, g_ref, b_ref,
                fc1b_ref, g1_ref, b1_ref, w2fc_ref, fc2b_ref, g2_ref, b2_ref,
                w3fc_ref, fc3b_ref, g3_ref, b3_ref,
                out_ref, wd_s, u_s, acc_ref):
    f32 = jnp.float32
    pid = pl.program_id(0)

    @pl.when(pid == 0)
    def _():
        cnt = jnp.sum(cnt_ref[...], axis=0)
        cntT = jnp.sum(cntT_ref[...], axis=0)
        a = adj_ref[...]
        sym = jnp.maximum(a, jnp.swapaxes(a, 1, 2))
        s_mat = jnp.sum(sym, axis=0)

        ii = lax.broadcasted_iota(jnp.int32, (_NN, _NN), 0)
        jj = lax.broadcasted_iota(jnp.int32, (_NN, _NN), 1)
        offdiag = ii != jj
        c_mat = jnp.where(offdiag, cnt * s_mat, 0.0)
        ct_mat = jnp.where(offdiag, cntT * s_mat, 0.0)

        deg = jnp.sum(c_mat, axis=1)
        safe = jnp.where(deg > 0, deg, 1.0)
        dis = jnp.where(deg > 0, lax.rsqrt(safe), 0.0)[:, None]

        x0 = x0_ref[...]
        u1 = jnp.dot(ct_mat, dis * x0, preferred_element_type=f32)
        m1 = -dis * u1
        u2 = jnp.dot(ct_mat, dis * m1, preferred_element_type=f32)
        m2 = -dis * u2
        u_s[...] = (jnp.dot(m1, w1_ref[...], preferred_element_type=f32)
                    + 2.0 * jnp.dot(m2, w2_ref[...], preferred_element_type=f32))
        wd_s[...] = w0_ref[...] - w2_ref[...]
        acc_ref[...] = jnp.zeros_like(acc_ref)

    bf16 = jnp.bfloat16
    wd = wd_s[...].astype(bf16)
    cheb = cheb_ref[...]
    g = g_ref[...]
    b = b_ref[...]
    row0 = (lax.broadcasted_iota(jnp.int32, (_B, 1), 0) == 0).astype(f32)
    acc = acc_ref[...]
    for j in range(_KN):
        xj = xr_ref[:, j, :].astype(bf16)
        yj = jnp.dot(xj, wd, preferred_element_type=f32)
        uj = u_s[pl.ds(pid * _KN + j, 1), :]
        yj = yj + row0 * uj + cheb
        actj = jax.nn.softplus(yj) * (_INV * g) + b
        acc = acc + lax.dot_general(
            actj.astype(bf16), wf_ref[:, j, :].astype(bf16),
            (((1,), (1,)), ((), ())),
            preferred_element_type=f32)
    acc_ref[...] = acc

    @pl.when(pid == pl.num_programs(0) - 1)
    def _():
        h = acc_ref[...] + fc1b_ref[...]
        h = jax.nn.softplus(h) * (_INV * g1_ref[...]) + b1_ref[...]
        h = lax.dot_general(h, w2fc_ref[...], (((1,), (1,)), ((), ())),
                            preferred_element_type=f32) + fc2b_ref[...]
        h = jax.nn.softplus(h) * (_INV * g2_ref[...]) + b2_ref[...]
        h = lax.dot_general(h, w3fc_ref[...], (((1,), (1,)), ((), ())),
                            preferred_element_type=f32) + fc3b_ref[...]
        h = h * (_INV * g3_ref[...]) + b3_ref[...]
        h = jax.nn.softplus(h)
        m = jnp.max(h, axis=1, keepdims=True)
        e = jnp.exp(h - m)
        out_ref[...] = e / jnp.sum(e, axis=1, keepdims=True)


def kernel(input, adj_weights, edge_index, W0, W1, W2, cheb_b, bn1_g, bn1_b,
           fc1_W, fc1_b, bnf1_g, bnf1_b, fc2_W, fc2_b, bnf2_g, bnf2_b,
           fc3_W, fc3_b, bnf3_g, bnf3_b):
    f32 = jnp.float32
    x0 = input[0]
    e = edge_index.shape[1]
    src = jnp.pad(edge_index[0], (0, _EP - e))
    dst = jnp.pad(edge_index[1], (0, _EP - e))

    cnt_parts, cntT_parts = _edge_hist(src, dst)
    cnt_parts = cnt_parts.reshape(_NW, _NN, _NN)
    cntT_parts = cntT_parts.reshape(_NW, _NN, _NN)

    n_k = _NN // _KN
    const = lambda shape: pl.BlockSpec(shape, lambda i: tuple(0 for _ in shape))
    out = pl.pallas_call(
        _fused_body,
        grid=(n_k,),
        in_specs=[
            const((_NW, _NN, _NN)),
            const((_NW, _NN, _NN)),
            const((_B, _NN, _NN)),
            const((_NN, _F)),
            const((_F, _F)),
            const((_F, _F)),
            const((_F, _F)),
            pl.BlockSpec((_B, _KN, _F), lambda i: (0, i, 0)),
            pl.BlockSpec((256, _KN, _F), lambda i: (0, i, 0)),
            const((1, _F)),
            const((1, _F)),
            const((1, _F)),
            const((1, 256)),
            const((1, 256)),
            const((1, 256)),
            const((32, 256)),
            const((1, 32)),
            const((1, 32)),
            const((1, 32)),
            const((4, 32)),
            const((1, 4)),
            const((1, 4)),
            const((1, 4)),
        ],
        out_specs=pl.BlockSpec((_B, 4), lambda i: (0, 0)),
        out_shape=jax.ShapeDtypeStruct((_B, 4), f32),
        scratch_shapes=[
            pltpu.VMEM((_F, _F), f32),
            pltpu.VMEM((_NN, _F), f32),
            pltpu.VMEM((_B, 256), f32),
        ],
    )(cnt_parts, cntT_parts, adj_weights, x0, W0, W1, W2,
      input, fc1_W.reshape(256, _NN, _F),
      cheb_b.reshape(1, _F), bn1_g.reshape(1, _F), bn1_b.reshape(1, _F),
      fc1_b.reshape(1, 256), bnf1_g.reshape(1, 256), bnf1_b.reshape(1, 256),
      fc2_W, fc2_b.reshape(1, 32), bnf2_g.reshape(1, 32), bnf2_b.reshape(1, 32),
      fc3_W, fc3_b.reshape(1, 4), bnf3_g.reshape(1, 4), bnf3_b.reshape(1, 4))
    return out


# speedup vs baseline: 2.0198x; 2.0198x over previous
import functools

import jax
import jax.numpy as jnp
from jax import lax
from jax.experimental import pallas as pl
from jax.experimental.pallas import tpu as pltpu
from jax.experimental.pallas import tpu_sc as plsc

_NN = 64
_B = 64
_F = 657
_EP = 4096
_NW = 32
_EPW = _EP // _NW
_NBUF = 3
_INV = 0.9999950000374997


_sc_mesh = plsc.VectorSubcoreMesh(core_axis_name="c", subcore_axis_name="s")


@functools.partial(
    pl.kernel,
    mesh=_sc_mesh,
    out_type=[
        jax.ShapeDtypeStruct((_NW, _NN * _NN), jnp.float32),
        jax.ShapeDtypeStruct((_NW, _NN * _NN), jnp.float32),
    ],
    scratch_types=[
        pltpu.VMEM((_EPW,), jnp.int32),
        pltpu.VMEM((_EPW,), jnp.int32),
        pltpu.VMEM((_NN * _NN,), jnp.float32),
        pltpu.VMEM((_NN * _NN,), jnp.float32),
    ],
    compiler_params=pltpu.CompilerParams(needs_layout_passes=False),
)
def _edge_hist(src_hbm, dst_hbm, out_hbm, outT_hbm, src_v, dst_v, cnt_v, cntT_v):
    c = lax.axis_index("c")
    s = lax.axis_index("s")
    wid = s * 2 + c
    base = wid * _EPW
    pltpu.sync_copy(src_hbm.at[pl.ds(base, _EPW)], src_v)
    pltpu.sync_copy(dst_hbm.at[pl.ds(base, _EPW)], dst_v)

    zeros16 = jnp.zeros((16,), jnp.float32)

    def _zero(i, carry):
        cnt_v[pl.ds(i * 16, 16)] = zeros16
        cntT_v[pl.ds(i * 16, 16)] = zeros16
        return carry

    lax.fori_loop(0, (_NN * _NN) // 16, _zero, 0)

    ones16 = jnp.ones((16,), jnp.float32)

    def _scatter(j, carry):
        sv = src_v[pl.ds(j * 16, 16)]
        dv = dst_v[pl.ds(j * 16, 16)]
        plsc.addupdate_scatter(cnt_v, [sv * _NN + dv], ones16)
        plsc.addupdate_scatter(cntT_v, [dv * _NN + sv], ones16)
        return carry

    lax.fori_loop(0, _EPW // 16, _scatter, 0)

    pltpu.sync_copy(cnt_v, out_hbm.at[wid])
    pltpu.sync_copy(cntT_v, outT_hbm.at[wid])


def _wf_copy(wf_hbm, wf_s, sem, c):
    half = 128
    return pltpu.make_async_copy(
        wf_hbm.at[pl.ds(c * half, half), :], wf_s.at[pl.ds(c * half, half), :],
        sem.at[c])


def _x_copy(x_hbm, x_buf, xsem, k):
    return pltpu.make_async_copy(
        x_hbm.at[:, pl.ds(k * 8, 8), :], x_buf.at[k % _NBUF],
        xsem.at[k % _NBUF])


def _fused_body(cnt_ref, cntT_ref, adj_ref, x0_ref, w0_ref, w1_ref, w2_ref,
                x_hbm, wf_hbm, cheb_ref, g_ref, b_ref,
                fc1b_ref, g1_ref, b1_ref, w2fc_ref, fc2b_ref, g2_ref, b2_ref,
                w3fc_ref, fc3b_ref, g3_ref, b3_ref,
                out_ref, wf_s, x_buf, sem, xsem):
    f32 = jnp.float32

    for c in range(2):
        _wf_copy(wf_hbm, wf_s, sem, c).start()
    for k in range(_NBUF - 1):
        _x_copy(x_hbm, x_buf, xsem, k).start()

    cnt = jnp.sum(cnt_ref[...], axis=0)
    cntT = jnp.sum(cntT_ref[...], axis=0)
    a = adj_ref[...]
    sym = jnp.maximum(a, jnp.swapaxes(a, 1, 2))
    s_mat = jnp.sum(sym, axis=0)

    ii = lax.broadcasted_iota(jnp.int32, (_NN, _NN), 0)
    jj = lax.broadcasted_iota(jnp.int32, (_NN, _NN), 1)
    offdiag = ii != jj
    c_mat = jnp.where(offdiag, cnt * s_mat, 0.0)
    ct_mat = jnp.where(offdiag, cntT * s_mat, 0.0)

    deg = jnp.sum(c_mat, axis=1)
    safe = jnp.where(deg > 0, deg, 1.0)
    dis = jnp.where(deg > 0, lax.rsqrt(safe), 0.0)[:, None]

    x0 = x0_ref[...]
    u1 = jnp.dot(ct_mat, dis * x0, preferred_element_type=f32)
    m1 = -dis * u1
    u2 = jnp.dot(ct_mat, dis * m1, preferred_element_type=f32)
    m2 = -dis * u2
    u = (jnp.dot(m1, w1_ref[...], preferred_element_type=f32)
         + 2.0 * jnp.dot(m2, w2_ref[...], preferred_element_type=f32))
    wd = w0_ref[...] - w2_ref[...]

    cheb = cheb_ref[...]
    g = g_ref[...]
    b = b_ref[...]
    row0 = (lax.broadcasted_iota(jnp.int32, (_B, 1), 0) == 0).astype(f32)
    for c in range(2):
        _wf_copy(wf_hbm, wf_s, sem, c).wait()
    acc = jnp.zeros((_B, 256), f32)
    for k in range(_NN // 8):
        _x_copy(x_hbm, x_buf, xsem, k).wait()
        if k + _NBUF - 1 < _NN // 8:
            _x_copy(x_hbm, x_buf, xsem, k + _NBUF - 1).start()
        for jj in range(8):
            j = k * 8 + jj
            xj = x_buf[k % _NBUF][:, jj, :]
            yj = jnp.dot(xj, wd, preferred_element_type=f32)
            yj = yj + row0 * u[j, :][None, :] + cheb
            actj = jax.nn.softplus(yj) * (_INV * g) + b
            acc = acc + lax.dot_general(
                actj, wf_s[:, j * _F:(j + 1) * _F],
                (((1,), (1,)), ((), ())),
                preferred_element_type=f32)

    h = acc + fc1b_ref[...]
    h = jax.nn.softplus(h) * (_INV * g1_ref[...]) + b1_ref[...]
    h = lax.dot_general(h, w2fc_ref[...], (((1,), (1,)), ((), ())),
                        preferred_element_type=f32) + fc2b_ref[...]
    h = jax.nn.softplus(h) * (_INV * g2_ref[...]) + b2_ref[...]
    h = lax.dot_general(h, w3fc_ref[...], (((1,), (1,)), ((), ())),
                        preferred_element_type=f32) + fc3b_ref[...]
    h = h * (_INV * g3_ref[...]) + b3_ref[...]
    h = jax.nn.softplus(h)
    m = jnp.max(h, axis=1, keepdims=True)
    e = jnp.exp(h - m)
    out_ref[...] = e / jnp.sum(e, axis=1, keepdims=True)


def kernel(input, adj_weights, edge_index, W0, W1, W2, cheb_b, bn1_g, bn1_b,
           fc1_W, fc1_b, bnf1_g, bnf1_b, fc2_W, fc2_b, bnf2_g, bnf2_b,
           fc3_W, fc3_b, bnf3_g, bnf3_b):
    f32 = jnp.float32
    x0 = input[0]
    e = edge_index.shape[1]
    src = jnp.pad(edge_index[0], (0, _EP - e))
    dst = jnp.pad(edge_index[1], (0, _EP - e))

    cnt_parts, cntT_parts = _edge_hist(src, dst)
    cnt_parts = cnt_parts.reshape(_NW, _NN, _NN)
    cntT_parts = cntT_parts.reshape(_NW, _NN, _NN)

    vmem = lambda: pl.BlockSpec(memory_space=pltpu.MemorySpace.VMEM)
    out = pl.pallas_call(
        _fused_body,
        in_specs=[
            vmem(),
            vmem(),
            vmem(),
            vmem(),
            vmem(),
            vmem(),
            vmem(),
            pl.BlockSpec(memory_space=pltpu.MemorySpace.HBM),
            pl.BlockSpec(memory_space=pltpu.MemorySpace.HBM),
            vmem(),
            vmem(),
            vmem(),
            vmem(),
            vmem(),
            vmem(),
            vmem(),
            vmem(),
            vmem(),
            vmem(),
            vmem(),
            vmem(),
            vmem(),
            vmem(),
        ],
        out_specs=pl.BlockSpec(memory_space=pltpu.MemorySpace.VMEM),
        out_shape=jax.ShapeDtypeStruct((_B, 4), f32),
        scratch_shapes=[
            pltpu.VMEM((256, _NN * _F), f32),
            pltpu.VMEM((_NBUF, _B, 8, _F), f32),
            pltpu.SemaphoreType.DMA((2,)),
            pltpu.SemaphoreType.DMA((_NBUF,)),
        ],
        compiler_params=pltpu.CompilerParams(
            vmem_limit_bytes=100 * 1024 * 1024),
    )(cnt_parts, cntT_parts, adj_weights, x0, W0, W1, W2,
      input, fc1_W,
      cheb_b.reshape(1, _F), bn1_g.reshape(1, _F), bn1_b.reshape(1, _F),
      fc1_b.reshape(1, 256), bnf1_g.reshape(1, 256), bnf1_b.reshape(1, 256),
      fc2_W, fc2_b.reshape(1, 32), bnf2_g.reshape(1, 32), bnf2_b.reshape(1, 32),
      fc3_W, fc3_b.reshape(1, 4), bnf3_g.reshape(1, 4), bnf3_b.reshape(1, 4))
    return out
